# baseline (speedup 1.0000x reference)
import functools

import jax
import jax.numpy as jnp
from jax import lax
from jax.experimental import pallas as pl
from jax.experimental.pallas import tpu as pltpu
from jax.experimental.pallas import tpu_sc as plsc

_NC = 2
_NS = 16
_L = 16

_MESH = plsc.VectorSubcoreMesh(core_axis_name="c", subcore_axis_name="s")
_SC_PARAMS = pltpu.CompilerParams(
    needs_layout_passes=False, use_tc_tiling_on_sc=False
)

_DOT = functools.partial(
    jnp.dot, preferred_element_type=jnp.float32, precision=lax.Precision.HIGHEST
)


def _deg_call(row, w, n_pad):
  e = row.shape[0]
  ec = e // _NC
  et = ec // _NS
  ch = 80
  nblk = 2000
  n_blocks = et // nblk
  nsub = nblk // ch
  rpt = n_pad // _NS

  @functools.partial(
      pl.kernel,
      out_type=jax.ShapeDtypeStruct((_NC, n_pad), jnp.float32),
      mesh=_MESH,
      compiler_params=_SC_PARAMS,
      scratch_types=[
          pltpu.VMEM_SHARED((n_pad,), jnp.float32),
          pltpu.VMEM((nblk,), jnp.int32),
          pltpu.VMEM((nblk,), jnp.float32),
          pltpu.VMEM((ch,), jnp.int32),
          pltpu.VMEM((ch,), jnp.int32),
          pltpu.VMEM((rpt,), jnp.float32),
          pltpu.SemaphoreType.DMA,
          pltpu.SemaphoreType.DMA,
      ],
  )
  def deg_kernel(
      row_hbm, w_hbm, out_hbm, acc, rowb, wb, rowi_a, rowi_b, zb, sem_a, sem_b
  ):
    c = lax.axis_index("c")
    s = lax.axis_index("s")

    @pl.loop(0, rpt // _L)
    def _(i):
      zb[pl.ds(i * _L, _L)] = jnp.zeros((_L,), jnp.float32)

    pltpu.sync_copy(zb, acc.at[pl.ds(s * rpt, rpt)])
    plsc.subcore_barrier()
    ebase = c * ec + s * et

    def build(e0, rowi):
      for j in range(ch // _L):
        rowi[pl.ds(j * _L, _L)] = rowb[pl.ds(e0 + j * _L, _L)]

    @pl.loop(0, n_blocks)
    def _(blk):
      b0 = ebase + blk * nblk
      pltpu.sync_copy(row_hbm.at[pl.ds(b0, nblk)], rowb)
      pltpu.sync_copy(w_hbm.at[pl.ds(b0, nblk)], wb)

      @pl.loop(0, nsub // 2)
      def _(pair):
        e_a = pair * 2 * ch
        e_b = e_a + ch
        build(e_a, rowi_a)
        s_a = pltpu.async_copy(wb.at[pl.ds(e_a, ch)], acc.at[rowi_a], sem_a, add=True)
        build(e_b, rowi_b)
        s_b = pltpu.async_copy(wb.at[pl.ds(e_b, ch)], acc.at[rowi_b], sem_b, add=True)
        s_a.wait()
        s_b.wait()

      if nsub % 2:
        e0 = (nsub - 1) * ch
        build(e0, rowi_a)
        pltpu.async_copy(wb.at[pl.ds(e0, ch)], acc.at[rowi_a], sem_a, add=True).wait()

    plsc.subcore_barrier()
    pltpu.sync_copy(acc.at[pl.ds(s * rpt, rpt)], out_hbm.at[c, pl.ds(s * rpt, rpt)])

  return deg_kernel(row, w)


def _stats_call(degp):
  np_ = degp.shape[1]

  def body(d_ref, dis_ref, sw_ref):
    d = d_ref[0:1, :] + d_ref[1:2, :] + 1.0
    dis_ref[...] = lax.rsqrt(d)
    sw_ref[...] = 1.0 / d

  return pl.pallas_call(
      body,
      grid=(1,),
      in_specs=[pl.BlockSpec((_NC, np_), lambda i: (0, 0))],
      out_specs=[
          pl.BlockSpec((1, np_), lambda i: (0, 0)),
          pl.BlockSpec((1, np_), lambda i: (0, 0)),
      ],
      out_shape=[
          jax.ShapeDtypeStruct((1, np_), jnp.float32),
          jax.ShapeDtypeStruct((1, np_), jnp.float32),
      ],
  )(degp)


def _wnorm_call(row, col, w, dis_pad):
  e = row.shape[0]
  np_ = dis_pad.shape[0]
  ec = e // _NC
  et = ec // _NS
  ch = 2000
  n_chunks = et // ch

  @functools.partial(
      pl.kernel,
      out_type=jax.ShapeDtypeStruct((e,), jnp.float32),
      mesh=_MESH,
      compiler_params=_SC_PARAMS,
      scratch_types=[
          pltpu.VMEM((np_,), jnp.float32),
          pltpu.VMEM((ch,), jnp.int32),
          pltpu.VMEM((ch,), jnp.int32),
          pltpu.VMEM((ch,), jnp.float32),
          pltpu.VMEM((ch,), jnp.float32),
      ],
  )
  def wn_kernel(row_hbm, col_hbm, w_hbm, dis_hbm, out_hbm, disv, rowb, colb, wb, wnb):
    c = lax.axis_index("c")
    s = lax.axis_index("s")
    pltpu.sync_copy(dis_hbm, disv)
    ebase = c * ec + s * et

    @pl.loop(0, n_chunks)
    def _(i):
      b = ebase + i * ch
      pltpu.sync_copy(row_hbm.at[pl.ds(b, ch)], rowb)
      pltpu.sync_copy(col_hbm.at[pl.ds(b, ch)], colb)
      pltpu.sync_copy(w_hbm.at[pl.ds(b, ch)], wb)

      @pl.loop(0, ch // _L)
      def _(j):
        sl = pl.ds(j * _L, _L)
        dr = plsc.load_gather(disv, [rowb[sl]])
        dc = plsc.load_gather(disv, [colb[sl]])
        wnb[sl] = dr * wb[sl] * dc

      pltpu.sync_copy(wnb, out_hbm.at[pl.ds(b, ch)])

  return wn_kernel(row, col, w, dis_pad)


def _prop_call(table, row, col, wn, n, dw, feat_split):
  e = row.shape[0]
  ec = e if feat_split else e // _NC
  et = ec // _NS
  nblk = 2000
  n_blocks = et // nblk
  ch = 80
  nsub = nblk // ch
  rpt = n // _NS
  zr = 25

  @functools.partial(
      pl.kernel,
      out_type=jax.ShapeDtypeStruct((_NC, n, dw), jnp.float32),
      mesh=_MESH,
      compiler_params=_SC_PARAMS,
      scratch_types=[
          pltpu.VMEM_SHARED((n, dw), jnp.float32),
          pltpu.VMEM((nblk,), jnp.int32),
          pltpu.VMEM((nblk,), jnp.int32),
          pltpu.VMEM((nblk,), jnp.float32),
          pltpu.VMEM((ch,), jnp.int32),
          pltpu.VMEM((ch,), jnp.int32),
          pltpu.VMEM((ch,), jnp.int32),
          pltpu.VMEM((ch,), jnp.int32),
          pltpu.VMEM((ch,), jnp.int32),
          pltpu.VMEM((ch,), jnp.int32),
          pltpu.VMEM((ch,), jnp.int32),
          pltpu.VMEM((ch,), jnp.int32),
          pltpu.VMEM((ch, dw), jnp.float32),
          pltpu.VMEM((ch, dw), jnp.float32),
          pltpu.VMEM((ch, dw), jnp.float32),
          pltpu.VMEM((ch, dw), jnp.float32),
          pltpu.VMEM((zr, dw), jnp.float32),
          pltpu.SemaphoreType.DMA,
          pltpu.SemaphoreType.DMA,
          pltpu.SemaphoreType.DMA,
          pltpu.SemaphoreType.DMA,
          pltpu.SemaphoreType.DMA,
          pltpu.SemaphoreType.DMA,
          pltpu.SemaphoreType.DMA,
          pltpu.SemaphoreType.DMA,
      ],
  )
  def prop_kernel(
      table_hbm, row_hbm, col_hbm, wn_hbm, out_hbm,
      acc, rowb, colb, wnb,
      rowi_0, coli_0, rowi_1, coli_1, rowi_2, coli_2, rowi_3, coli_3,
      rows_0, rows_1, rows_2, rows_3,
      zbuf, gsem_0, gsem_1, gsem_2, gsem_3, ssem_0, ssem_1, ssem_2, ssem_3,
  ):
    c = lax.axis_index("c")
    s = lax.axis_index("s")

    @pl.loop(0, zr)
    def _(r):
      for v in range(dw // _L):
        zbuf[r, pl.ds(v * _L, _L)] = jnp.zeros((_L,), jnp.float32)

    @pl.loop(0, rpt // zr)
    def _(z):
      pltpu.sync_copy(zbuf, acc.at[pl.ds(s * rpt + z * zr, zr)])

    plsc.subcore_barrier()
    ebase = (0 if feat_split else c * ec) + s * et
    roff = c * n if feat_split else 0

    def build(e0, rowi, coli):
      for j in range(ch // _L):
        sl_s = pl.ds(j * _L, _L)
        sl_b = pl.ds(e0 + j * _L, _L)
        rowi[sl_s] = rowb[sl_b] + roff
        coli[sl_s] = colb[sl_b]

    def scale(e0, rows):
      @plsc.parallel_loop(0, ch // _L)
      def _(g):
        wv16 = wnb[pl.ds(e0 + g * _L, _L)]
        for jj in range(_L):
          wv = wv16[jj]
          j = g * _L + jj
          for v in range(dw // _L):
            sl = pl.ds(v * _L, _L)
            rows[j, sl] = rows[j, sl] * wv

    idxs = ((rowi_0, coli_0), (rowi_1, coli_1), (rowi_2, coli_2), (rowi_3, coli_3))
    rows = (rows_0, rows_1, rows_2, rows_3)
    gsems = (gsem_0, gsem_1, gsem_2, gsem_3)
    ssems = (ssem_0, ssem_1, ssem_2, ssem_3)

    def issue_gather(e0, bi):
      build(e0, *idxs[bi])
      return pltpu.async_copy(table_hbm.at[idxs[bi][0]], rows[bi], gsems[bi])

    def wait_gather(bi):
      pltpu.make_async_copy(table_hbm.at[idxs[bi][0]], rows[bi], gsems[bi]).wait()

    def consume(e0, bi):
      scale(e0, rows[bi])
      return pltpu.async_copy(rows[bi], acc.at[idxs[bi][1]], ssems[bi], add=True)

    def wait_scatter(bi):
      pltpu.make_async_copy(rows[bi], acc.at[idxs[bi][1]], ssems[bi]).wait()

    niter = nsub // 4
    n_tail = nsub - 4 * niter

    @pl.loop(0, n_blocks)
    def _(blk):
      b0 = ebase + blk * nblk
      pltpu.sync_copy(row_hbm.at[pl.ds(b0, nblk)], rowb)
      pltpu.sync_copy(col_hbm.at[pl.ds(b0, nblk)], colb)
      pltpu.sync_copy(wn_hbm.at[pl.ds(b0, nblk)], wnb)

      @pl.when(blk > 0)
      def _():
        wait_scatter(2)
        wait_scatter(3)

      issue_gather(0, 0)
      issue_gather(ch, 1)

      @pl.loop(0, niter)
      def _(k):
        @pl.when(k > 0)
        def _():
          wait_scatter(2)
          wait_scatter(3)

        e0 = k * 4 * ch
        g2 = issue_gather(e0 + 2 * ch, 2)
        g3 = issue_gather(e0 + 3 * ch, 3)
        wait_gather(0)
        s0 = consume(e0, 0)
        wait_gather(1)
        s1 = consume(e0 + ch, 1)
        g2.wait()
        consume(e0 + 2 * ch, 2)
        g3.wait()
        consume(e0 + 3 * ch, 3)
        s0.wait()
        s1.wait()

        @pl.when(k < niter - 1)
        def _():
          issue_gather(e0 + 4 * ch, 0)
          issue_gather(e0 + 5 * ch, 1)

      for t in range(n_tail):
        e0 = (4 * niter + t) * ch
        bi = t
        g = issue_gather(e0, bi)
        g.wait()
        consume(e0, bi).wait()

    wait_scatter(2)
    wait_scatter(3)
    plsc.subcore_barrier()

    @pl.loop(0, rpt // zr)
    def _(z):
      r0 = s * rpt + z * zr
      pltpu.sync_copy(acc.at[pl.ds(r0, zr)], out_hbm.at[c, pl.ds(r0, zr)])

  return prop_kernel(table, row, col, wn)


def _mm_call(x, w0):
  n, d = x.shape
  h = w0.shape[1]
  rb = 2000
  n_i = n // rb
  n_j = h // 128

  def body(x_ref, w_ref, o_ref):
    o_ref[...] = _DOT(x_ref[...], w_ref[...])

  return pl.pallas_call(
      body,
      grid=(n_i, n_j),
      in_specs=[
          pl.BlockSpec((rb, d), lambda i, j: (i, 0)),
          pl.BlockSpec((d, 128), lambda i, j: (0, j)),
      ],
      out_specs=pl.BlockSpec((rb, 128), lambda i, j: (j * n_i + i, 0)),
      out_shape=jax.ShapeDtypeStruct((n_j * n, 128), jnp.float32),
  )(x, w0)


def _fused_call(s2, hr, selfw, b, w, n_j):
  n = s2.shape[1]
  rb = 2000
  n_i = n // rb
  ob = w.shape[1] // n_j

  def body(s_ref, h_ref, sw_ref, b_ref, w_ref, o_ref):
    sw = sw_ref[...]
    bb = b_ref[...]
    s = s_ref[...]
    hh = h_ref[...]
    p0 = jnp.maximum(s[0] + sw * hh[0] + bb[:, :128], 0.0)
    p1 = jnp.maximum(s[1] + sw * hh[1] + bb[:, 128:], 0.0)
    ww = w_ref[...]
    o_ref[...] = _DOT(p0, ww[:128]) + _DOT(p1, ww[128:])

  return pl.pallas_call(
      body,
      grid=(n_i, n_j),
      in_specs=[
          pl.BlockSpec((2, rb, 128), lambda i, j: (0, i, 0)),
          pl.BlockSpec((2, rb, 128), lambda i, j: (0, i, 0)),
          pl.BlockSpec((rb, 1), lambda i, j: (i, 0)),
          pl.BlockSpec((1, 256), lambda i, j: (0, 0)),
          pl.BlockSpec((256, ob), lambda i, j: (0, j)),
      ],
      out_specs=pl.BlockSpec((rb, ob), lambda i, j: (j * n_i + i, 0)),
      out_shape=jax.ShapeDtypeStruct((n_j * n, ob), jnp.float32),
  )(s2, hr, selfw, b, w)


def _final_call(s2, h2, selfw, b2p, n_out):
  n, dw = h2.shape
  rb = 2000
  n_i = n // rb

  def body(s_ref, h_ref, sw_ref, b_ref, o_ref):
    s = s_ref[...]
    full = s[0] + s[1] + sw_ref[...] * h_ref[...] + b_ref[...]
    o_ref[...] = full[:, :n_out]

  return pl.pallas_call(
      body,
      grid=(n_i,),
      in_specs=[
          pl.BlockSpec((2, rb, dw), lambda i: (0, i, 0)),
          pl.BlockSpec((rb, dw), lambda i: (i, 0)),
          pl.BlockSpec((rb, 1), lambda i: (i, 0)),
          pl.BlockSpec((1, dw), lambda i: (0, 0)),
      ],
      out_specs=pl.BlockSpec((rb, n_out), lambda i: (i, 0)),
      out_shape=jax.ShapeDtypeStruct((n, n_out), jnp.float32),
  )(s2, h2, selfw, b2p)


def kernel(x, edge_index, edge_weight, W0, b0, W1, b1, W2, b2):
  n = x.shape[0]
  n_pad = 10240
  c_out = W2.shape[1]

  row = edge_index[0]
  col = edge_index[1]

  degp = _deg_call(row, edge_weight, n_pad)
  dis_p, sw_p = _stats_call(degp)
  dis = dis_p.reshape(n_pad)
  selfw = sw_p.reshape(n_pad)[:n].reshape(n, 1)

  wn = _wnorm_call(row, col, edge_weight, dis)

  h0 = _mm_call(x, W0)
  s0 = _prop_call(h0, row, col, wn, n, 128, True)
  h1 = _fused_call(s0, h0.reshape(2, n, 128), selfw, b0.reshape(1, -1), W1, 2)
  s1 = _prop_call(h1, row, col, wn, n, 128, True)

  w2p = jnp.pad(W2, ((0, 0), (0, 48 - c_out)))
  h2 = _fused_call(s1, h1.reshape(2, n, 128), selfw, b1.reshape(1, -1), w2p, 1)
  s2 = _prop_call(h2, row, col, wn, n, 48, False)

  b2p = jnp.pad(b2, (0, 48 - c_out)).reshape(1, -1)
  return _final_call(s2, h2, selfw, b2p, c_out)


# speedup vs baseline: 1.3148x; 1.3148x over previous
import functools

import jax
import jax.numpy as jnp
from jax import lax
from jax.experimental import pallas as pl
from jax.experimental.pallas import tpu as pltpu
from jax.experimental.pallas import tpu_sc as plsc

_NC = 2
_NS = 16
_L = 16

_MESH = plsc.VectorSubcoreMesh(core_axis_name="c", subcore_axis_name="s")
_SC_PARAMS = pltpu.CompilerParams(
    needs_layout_passes=False, use_tc_tiling_on_sc=False
)

_DOT = functools.partial(
    jnp.dot, preferred_element_type=jnp.float32, precision=lax.Precision.HIGHEST
)


def _deg_call(row, w, n_pad):
  e = row.shape[0]
  ec = e // _NC
  et = ec // _NS
  ch = 80
  nblk = 2000
  n_blocks = et // nblk
  nsub = nblk // ch
  rpt = n_pad // _NS

  @functools.partial(
      pl.kernel,
      out_type=jax.ShapeDtypeStruct((_NC, n_pad), jnp.float32),
      mesh=_MESH,
      compiler_params=_SC_PARAMS,
      scratch_types=[
          pltpu.VMEM_SHARED((n_pad,), jnp.float32),
          pltpu.VMEM((nblk,), jnp.int32),
          pltpu.VMEM((nblk,), jnp.float32),
          pltpu.VMEM((ch,), jnp.int32),
          pltpu.VMEM((ch,), jnp.int32),
          pltpu.VMEM((rpt,), jnp.float32),
          pltpu.SemaphoreType.DMA,
          pltpu.SemaphoreType.DMA,
      ],
  )
  def deg_kernel(
      row_hbm, w_hbm, out_hbm, acc, rowb, wb, rowi_a, rowi_b, zb, sem_a, sem_b
  ):
    c = lax.axis_index("c")
    s = lax.axis_index("s")

    @pl.loop(0, rpt // _L)
    def _(i):
      zb[pl.ds(i * _L, _L)] = jnp.zeros((_L,), jnp.float32)

    pltpu.sync_copy(zb, acc.at[pl.ds(s * rpt, rpt)])
    plsc.subcore_barrier()
    ebase = c * ec + s * et

    def build(e0, rowi):
      for j in range(ch // _L):
        rowi[pl.ds(j * _L, _L)] = rowb[pl.ds(e0 + j * _L, _L)]

    @pl.loop(0, n_blocks)
    def _(blk):
      b0 = ebase + blk * nblk
      pltpu.sync_copy(row_hbm.at[pl.ds(b0, nblk)], rowb)
      pltpu.sync_copy(w_hbm.at[pl.ds(b0, nblk)], wb)

      @pl.loop(0, nsub // 2)
      def _(pair):
        e_a = pair * 2 * ch
        e_b = e_a + ch
        build(e_a, rowi_a)
        s_a = pltpu.async_copy(wb.at[pl.ds(e_a, ch)], acc.at[rowi_a], sem_a, add=True)
        build(e_b, rowi_b)
        s_b = pltpu.async_copy(wb.at[pl.ds(e_b, ch)], acc.at[rowi_b], sem_b, add=True)
        s_a.wait()
        s_b.wait()

      if nsub % 2:
        e0 = (nsub - 1) * ch
        build(e0, rowi_a)
        pltpu.async_copy(wb.at[pl.ds(e0, ch)], acc.at[rowi_a], sem_a, add=True).wait()

    plsc.subcore_barrier()
    pltpu.sync_copy(acc.at[pl.ds(s * rpt, rpt)], out_hbm.at[c, pl.ds(s * rpt, rpt)])

  return deg_kernel(row, w)


def _stats_call(degp):
  np_ = degp.shape[1]

  def body(d_ref, dis_ref, sw_ref):
    d = d_ref[0:1, :] + d_ref[1:2, :] + 1.0
    dis_ref[...] = lax.rsqrt(d)
    sw_ref[...] = 1.0 / d

  return pl.pallas_call(
      body,
      grid=(1,),
      in_specs=[pl.BlockSpec((_NC, np_), lambda i: (0, 0))],
      out_specs=[
          pl.BlockSpec((1, np_), lambda i: (0, 0)),
          pl.BlockSpec((1, np_), lambda i: (0, 0)),
      ],
      out_shape=[
          jax.ShapeDtypeStruct((1, np_), jnp.float32),
          jax.ShapeDtypeStruct((1, np_), jnp.float32),
      ],
  )(degp)


def _wnorm_call(row, col, w, dis_pad):
  e = row.shape[0]
  np_ = dis_pad.shape[0]
  ec = e // _NC
  et = ec // _NS
  ch = 2000
  n_chunks = et // ch

  @functools.partial(
      pl.kernel,
      out_type=jax.ShapeDtypeStruct((e,), jnp.float32),
      mesh=_MESH,
      compiler_params=_SC_PARAMS,
      scratch_types=[
          pltpu.VMEM((np_,), jnp.float32),
          pltpu.VMEM((ch,), jnp.int32),
          pltpu.VMEM((ch,), jnp.int32),
          pltpu.VMEM((ch,), jnp.float32),
          pltpu.VMEM((ch,), jnp.float32),
      ],
  )
  def wn_kernel(row_hbm, col_hbm, w_hbm, dis_hbm, out_hbm, disv, rowb, colb, wb, wnb):
    c = lax.axis_index("c")
    s = lax.axis_index("s")
    pltpu.sync_copy(dis_hbm, disv)
    ebase = c * ec + s * et

    @pl.loop(0, n_chunks)
    def _(i):
      b = ebase + i * ch
      pltpu.sync_copy(row_hbm.at[pl.ds(b, ch)], rowb)
      pltpu.sync_copy(col_hbm.at[pl.ds(b, ch)], colb)
      pltpu.sync_copy(w_hbm.at[pl.ds(b, ch)], wb)

      @pl.loop(0, ch // _L)
      def _(j):
        sl = pl.ds(j * _L, _L)
        dr = plsc.load_gather(disv, [rowb[sl]])
        dc = plsc.load_gather(disv, [colb[sl]])
        wnb[sl] = dr * wb[sl] * dc

      pltpu.sync_copy(wnb, out_hbm.at[pl.ds(b, ch)])

  return wn_kernel(row, col, w, dis_pad)


def _prop_call(table, row, col, wn, n, dw, feat_split):
  e = row.shape[0]
  ec = e if feat_split else e // _NC
  et = ec // _NS
  nblk = 2000
  n_blocks = et // nblk
  ch = 80
  nsub = nblk // ch
  rpt = n // _NS
  zr = 25

  @functools.partial(
      pl.kernel,
      out_type=jax.ShapeDtypeStruct((_NC, n, dw), jnp.float32),
      mesh=_MESH,
      compiler_params=_SC_PARAMS,
      scratch_types=[
          pltpu.VMEM_SHARED((n, dw), jnp.float32),
          pltpu.VMEM((nblk,), jnp.int32),
          pltpu.VMEM((nblk,), jnp.int32),
          pltpu.VMEM((nblk,), jnp.float32),
          pltpu.VMEM((ch,), jnp.int32),
          pltpu.VMEM((ch,), jnp.int32),
          pltpu.VMEM((ch,), jnp.int32),
          pltpu.VMEM((ch,), jnp.int32),
          pltpu.VMEM((ch,), jnp.int32),
          pltpu.VMEM((ch,), jnp.int32),
          pltpu.VMEM((ch,), jnp.int32),
          pltpu.VMEM((ch,), jnp.int32),
          pltpu.VMEM((ch, dw), jnp.float32),
          pltpu.VMEM((ch, dw), jnp.float32),
          pltpu.VMEM((ch, dw), jnp.float32),
          pltpu.VMEM((ch, dw), jnp.float32),
          pltpu.VMEM((zr, dw), jnp.float32),
          pltpu.SemaphoreType.DMA,
          pltpu.SemaphoreType.DMA,
          pltpu.SemaphoreType.DMA,
          pltpu.SemaphoreType.DMA,
          pltpu.SemaphoreType.DMA,
          pltpu.SemaphoreType.DMA,
          pltpu.SemaphoreType.DMA,
          pltpu.SemaphoreType.DMA,
      ],
  )
  def prop_kernel(
      table_hbm, row_hbm, col_hbm, wn_hbm, out_hbm,
      acc, rowb, colb, wnb,
      rowi_0, coli_0, rowi_1, coli_1, rowi_2, coli_2, rowi_3, coli_3,
      rows_0, rows_1, rows_2, rows_3,
      zbuf, gsem_0, gsem_1, gsem_2, gsem_3, ssem_0, ssem_1, ssem_2, ssem_3,
  ):
    c = lax.axis_index("c")
    s = lax.axis_index("s")

    @pl.loop(0, zr)
    def _(r):
      for v in range(dw // _L):
        zbuf[r, pl.ds(v * _L, _L)] = jnp.zeros((_L,), jnp.float32)

    @pl.loop(0, rpt // zr)
    def _(z):
      pltpu.sync_copy(zbuf, acc.at[pl.ds(s * rpt + z * zr, zr)])

    plsc.subcore_barrier()
    ebase = (0 if feat_split else c * ec) + s * et
    roff = c * n if feat_split else 0

    def build(e0, rowi, coli):
      for j in range(ch // _L):
        sl_s = pl.ds(j * _L, _L)
        sl_b = pl.ds(e0 + j * _L, _L)
        rowi[sl_s] = rowb[sl_b] + roff
        coli[sl_s] = colb[sl_b]

    def scale(e0, rows):
      @pl.loop(0, ch // _L)
      def _(g):
        wv16 = wnb[pl.ds(e0 + g * _L, _L)]
        for jj in range(_L):
          wv = wv16[jj]
          j = g * _L + jj
          for v in range(dw // _L):
            sl = pl.ds(v * _L, _L)
            rows[j, sl] = rows[j, sl] * wv

    idxs = ((rowi_0, coli_0), (rowi_1, coli_1), (rowi_2, coli_2), (rowi_3, coli_3))
    rows = (rows_0, rows_1, rows_2, rows_3)
    gsems = (gsem_0, gsem_1, gsem_2, gsem_3)
    ssems = (ssem_0, ssem_1, ssem_2, ssem_3)

    def issue_gather(e0, bi):
      build(e0, *idxs[bi])
      return pltpu.async_copy(table_hbm.at[idxs[bi][0]], rows[bi], gsems[bi])

    def wait_gather(bi):
      pltpu.make_async_copy(table_hbm.at[idxs[bi][0]], rows[bi], gsems[bi]).wait()

    def consume(e0, bi):
      scale(e0, rows[bi])
      return pltpu.async_copy(rows[bi], acc.at[idxs[bi][1]], ssems[bi], add=True)

    def wait_scatter(bi):
      pltpu.make_async_copy(rows[bi], acc.at[idxs[bi][1]], ssems[bi]).wait()

    niter = nsub // 4
    n_tail = nsub - 4 * niter

    @pl.loop(0, n_blocks)
    def _(blk):
      b0 = ebase + blk * nblk
      pltpu.sync_copy(row_hbm.at[pl.ds(b0, nblk)], rowb)
      pltpu.sync_copy(col_hbm.at[pl.ds(b0, nblk)], colb)
      pltpu.sync_copy(wn_hbm.at[pl.ds(b0, nblk)], wnb)

      @pl.when(blk > 0)
      def _():
        wait_scatter(2)
        wait_scatter(3)

      issue_gather(0, 0)
      issue_gather(ch, 1)

      @pl.loop(0, niter)
      def _(k):
        @pl.when(k > 0)
        def _():
          wait_scatter(2)
          wait_scatter(3)

        e0 = k * 4 * ch
        g2 = issue_gather(e0 + 2 * ch, 2)
        g3 = issue_gather(e0 + 3 * ch, 3)
        wait_gather(0)
        s0 = consume(e0, 0)
        wait_gather(1)
        s1 = consume(e0 + ch, 1)
        s0.wait()
        s1.wait()

        @pl.when(k < niter - 1)
        def _():
          issue_gather(e0 + 4 * ch, 0)
          issue_gather(e0 + 5 * ch, 1)

        g2.wait()
        consume(e0 + 2 * ch, 2)
        g3.wait()
        consume(e0 + 3 * ch, 3)

      for t in range(n_tail):
        e0 = (4 * niter + t) * ch
        bi = t
        g = issue_gather(e0, bi)
        g.wait()
        consume(e0, bi).wait()

    wait_scatter(2)
    wait_scatter(3)
    plsc.subcore_barrier()

    @pl.loop(0, rpt // zr)
    def _(z):
      r0 = s * rpt + z * zr
      pltpu.sync_copy(acc.at[pl.ds(r0, zr)], out_hbm.at[c, pl.ds(r0, zr)])

  return prop_kernel(table, row, col, wn)


def _mm_call(x, w0):
  n, d = x.shape
  h = w0.shape[1]
  rb = 2000
  n_i = n // rb
  n_j = h // 128

  def body(x_ref, w_ref, o_ref):
    o_ref[...] = _DOT(x_ref[...], w_ref[...])

  return pl.pallas_call(
      body,
      grid=(n_i, n_j),
      in_specs=[
          pl.BlockSpec((rb, d), lambda i, j: (i, 0)),
          pl.BlockSpec((d, 128), lambda i, j: (0, j)),
      ],
      out_specs=pl.BlockSpec((rb, 128), lambda i, j: (j * n_i + i, 0)),
      out_shape=jax.ShapeDtypeStruct((n_j * n, 128), jnp.float32),
  )(x, w0)


def _fused_call(s2, hr, selfw, b, w, n_j):
  n = s2.shape[1]
  rb = 2000
  n_i = n // rb
  ob = w.shape[1] // n_j

  def body(s_ref, h_ref, sw_ref, b_ref, w_ref, o_ref):
    sw = sw_ref[...]
    bb = b_ref[...]
    s = s_ref[...]
    hh = h_ref[...]
    p0 = jnp.maximum(s[0] + sw * hh[0] + bb[:, :128], 0.0)
    p1 = jnp.maximum(s[1] + sw * hh[1] + bb[:, 128:], 0.0)
    ww = w_ref[...]
    o_ref[...] = _DOT(p0, ww[:128]) + _DOT(p1, ww[128:])

  return pl.pallas_call(
      body,
      grid=(n_i, n_j),
      in_specs=[
          pl.BlockSpec((2, rb, 128), lambda i, j: (0, i, 0)),
          pl.BlockSpec((2, rb, 128), lambda i, j: (0, i, 0)),
          pl.BlockSpec((rb, 1), lambda i, j: (i, 0)),
          pl.BlockSpec((1, 256), lambda i, j: (0, 0)),
          pl.BlockSpec((256, ob), lambda i, j: (0, j)),
      ],
      out_specs=pl.BlockSpec((rb, ob), lambda i, j: (j * n_i + i, 0)),
      out_shape=jax.ShapeDtypeStruct((n_j * n, ob), jnp.float32),
  )(s2, hr, selfw, b, w)


def _final_call(s2, h2, selfw, b2p, n_out):
  n, dw = h2.shape
  rb = 2000
  n_i = n // rb

  def body(s_ref, h_ref, sw_ref, b_ref, o_ref):
    s = s_ref[...]
    full = s[0] + s[1] + sw_ref[...] * h_ref[...] + b_ref[...]
    o_ref[...] = full[:, :n_out]

  return pl.pallas_call(
      body,
      grid=(n_i,),
      in_specs=[
          pl.BlockSpec((2, rb, dw), lambda i: (0, i, 0)),
          pl.BlockSpec((rb, dw), lambda i: (i, 0)),
          pl.BlockSpec((rb, 1), lambda i: (i, 0)),
          pl.BlockSpec((1, dw), lambda i: (0, 0)),
      ],
      out_specs=pl.BlockSpec((rb, n_out), lambda i: (i, 0)),
      out_shape=jax.ShapeDtypeStruct((n, n_out), jnp.float32),
  )(s2, h2, selfw, b2p)


def kernel(x, edge_index, edge_weight, W0, b0, W1, b1, W2, b2):
  n = x.shape[0]
  n_pad = 10240
  c_out = W2.shape[1]

  row = edge_index[0]
  col = edge_index[1]

  degp = _deg_call(row, edge_weight, n_pad)
  dis_p, sw_p = _stats_call(degp)
  dis = dis_p.reshape(n_pad)
  selfw = sw_p.reshape(n_pad)[:n].reshape(n, 1)

  wn = _wnorm_call(row, col, edge_weight, dis)

  h0 = _mm_call(x, W0)
  s0 = _prop_call(h0, row, col, wn, n, 128, True)
  h1 = _fused_call(s0, h0.reshape(2, n, 128), selfw, b0.reshape(1, -1), W1, 2)
  s1 = _prop_call(h1, row, col, wn, n, 128, True)

  w2p = jnp.pad(W2, ((0, 0), (0, 48 - c_out)))
  h2 = _fused_call(s1, h1.reshape(2, n, 128), selfw, b1.reshape(1, -1), w2p, 1)
  s2 = _prop_call(h2, row, col, wn, n, 48, False)

  b2p = jnp.pad(b2, (0, 48 - c_out)).reshape(1, -1)
  return _final_call(s2, h2, selfw, b2p, c_out)
